# Initial kernel scaffold; baseline (speedup 1.0000x reference)
import jax
import jax.numpy as jnp
from jax.experimental import pallas as pl


def kernel(x, edge_index, edge_weight, W_emb, b_emb, Wl0, bl0, Wr0, Wl1, bl1, Wr1, Wl2, bl2, Wr2, Wlo, blo, Wro):
    raise NotImplementedError("write your pallas kernel here")


# speedup vs baseline: 4.1568x; 4.1568x over previous
import functools

import jax
import jax.numpy as jnp
from jax import lax
from jax.experimental import pallas as pl
from jax.experimental.pallas import tpu as pltpu
from jax.experimental.pallas import tpu_sc as plsc

N = 50000
E = 800000
D_IN = 100
H = 64
HH = 32
C = 18

NUM_SC = 2
NUM_TILES = 16

E_PAD = 819200
EDGE_ROWS = E_PAD // 128
EPT = E_PAD // NUM_TILES
CHUNK = 512
SUB = 128
ACC_ROWS = 51200
ZPT = ACC_ROWS // NUM_TILES
DCH = 200
DCHUNKS = N // DCH

_mesh = plsc.VectorSubcoreMesh(core_axis_name="c", subcore_axis_name="s")


def _make_scatter(scale: bool):
  def body(*refs):
    if scale:
      h_hbm, src_hbm, dst_hbm, w_hbm, out_hbm, acc, srcv, dstv, wv, rows, sem = refs
    else:
      h_hbm, src_hbm, dst_hbm, out_hbm, acc, srcv, dstv, wv, rows, sem = refs
    cid = lax.axis_index("c")
    sid = lax.axis_index("s")
    zero16 = jnp.zeros((16,), jnp.float32)

    for i in range(SUB):
      rows[i, pl.ds(0, 16)] = zero16
      rows[i, pl.ds(16, 16)] = zero16
    plsc.subcore_barrier()

    def zero_acc(i, carry):
      pltpu.sync_copy(rows.at[pl.ds(0, SUB)],
                      acc.at[pl.ds(sid * ZPT + i * SUB, SUB)])
      return carry

    lax.fori_loop(0, ZPT // SUB, zero_acc, 0)
    plsc.subcore_barrier()

    coff = cid * N

    def chunk(i, carry):
      r0 = sid * (EPT // 128) + i * (CHUNK // 128)
      pltpu.sync_copy(src_hbm.at[cid, pl.ds(r0, CHUNK // 128)], srcv)
      pltpu.sync_copy(dst_hbm.at[pl.ds(r0, CHUNK // 128)], dstv)
      if scale:
        pltpu.sync_copy(w_hbm.at[pl.ds(sid * EPT + i * CHUNK, CHUNK)], wv)
      copies = [
          pltpu.async_copy(h_hbm.at[srcv.at[j]],
                           rows.at[pl.ds(j * SUB, SUB)], sem)
          for j in range(CHUNK // SUB)
      ]
      for cp in copies:
        cp.wait()
      if scale:
        def grp(g, c2):
          w16 = wv[pl.ds(g * 16, 16)]
          for u in range(16):
            k = g * 16 + u
            w = w16[u]
            rows[k, pl.ds(0, 16)] = rows[k, pl.ds(0, 16)] * w
            rows[k, pl.ds(16, 16)] = rows[k, pl.ds(16, 16)] * w
          return c2

        lax.fori_loop(0, CHUNK // 16, grp, 0)
        plsc.subcore_barrier()
      for j in range(CHUNK // SUB):
        pltpu.sync_copy(rows.at[pl.ds(j * SUB, SUB)],
                        acc.at[dstv.at[j]], add=True)
      return carry

    lax.fori_loop(0, EPT // CHUNK, chunk, 0)
    plsc.subcore_barrier()

    def drain(k, carry):
      idx = sid + k * NUM_TILES

      @pl.when(idx < DCHUNKS)
      def _():
        b = idx * DCH
        pltpu.sync_copy(acc.at[pl.ds(b, DCH)], rows.at[pl.ds(0, DCH)])
        pltpu.sync_copy(rows.at[pl.ds(0, DCH)],
                        out_hbm.at[pl.ds(coff + b, DCH)])

      return carry

    lax.fori_loop(0, pl.cdiv(DCHUNKS, NUM_TILES), drain, 0)

  return pl.kernel(
      body,
      out_type=jax.ShapeDtypeStruct((NUM_SC * N, HH), jnp.float32),
      mesh=_mesh,
      compiler_params=pltpu.CompilerParams(use_tc_tiling_on_sc=False),
      scratch_types=[
          pltpu.VMEM_SHARED((ACC_ROWS, HH), jnp.float32),
          pltpu.VMEM((CHUNK // 128, 128), jnp.int32),
          pltpu.VMEM((CHUNK // 128, 128), jnp.int32),
          pltpu.VMEM((CHUNK,), jnp.float32),
          pltpu.VMEM((CHUNK, HH), jnp.float32),
          pltpu.SemaphoreType.DMA,
      ],
  )


def _count_body(dst_hbm, out_hbm, acc, dstv, ones, buf):
  cid = lax.axis_index("c")
  sid = lax.axis_index("s")
  zero16 = jnp.zeros((16,), jnp.float32)
  one16 = jnp.ones((16,), jnp.float32)
  for i in range(SUB):
    ones[i, pl.ds(0, 16)] = one16
  for i in range(SUB):
    buf[i, pl.ds(0, 16)] = zero16
  plsc.subcore_barrier()

  def zero_acc(i, carry):
    pltpu.sync_copy(buf.at[pl.ds(0, SUB)],
                    acc.at[pl.ds(sid * ZPT + i * SUB, SUB)])
    return carry

  lax.fori_loop(0, ZPT // SUB, zero_acc, 0)
  plsc.subcore_barrier()

  rows_per_tile = EDGE_ROWS // (NUM_SC * NUM_TILES)

  def chunk(i, carry):
    r0 = cid * (EDGE_ROWS // NUM_SC) + sid * rows_per_tile + i * 8
    pltpu.sync_copy(dst_hbm.at[pl.ds(r0, 8)], dstv)
    for j in range(8):
      pltpu.sync_copy(ones, acc.at[dstv.at[j]], add=True)
    return carry

  lax.fori_loop(0, rows_per_tile // 8, chunk, 0)
  plsc.subcore_barrier()

  def drain(k, carry):
    idx = sid + k * NUM_TILES

    @pl.when(idx < DCHUNKS)
    def _():
      b = idx * DCH
      pltpu.sync_copy(acc.at[pl.ds(b, DCH)], buf.at[pl.ds(0, DCH)])
      pltpu.sync_copy(buf.at[pl.ds(0, DCH)],
                      out_hbm.at[pl.ds(cid * N + b, DCH)])

    return carry

  lax.fori_loop(0, pl.cdiv(DCHUNKS, NUM_TILES), drain, 0)


_count_kernel = pl.kernel(
    _count_body,
    out_type=jax.ShapeDtypeStruct((NUM_SC * N, 16), jnp.float32),
    mesh=_mesh,
    compiler_params=pltpu.CompilerParams(use_tc_tiling_on_sc=False),
    scratch_types=[
        pltpu.VMEM_SHARED((ACC_ROWS, 16), jnp.float32),
        pltpu.VMEM((8, 128), jnp.int32),
        pltpu.VMEM((SUB, 16), jnp.float32),
        pltpu.VMEM((DCH, 16), jnp.float32),
    ],
)

_scatter_w = _make_scatter(True)
_scatter_nw = _make_scatter(False)

BN = 2000


def _emb_body(x_ref, w_ref, b_ref, o_ref):
  h = jnp.dot(x_ref[...], w_ref[...], preferred_element_type=jnp.float32)
  h = jnp.maximum(h + b_ref[...], 0.0)
  o_ref[0] = h[:, :HH]
  o_ref[1] = h[:, HH:]


_emb_call = pl.pallas_call(
    _emb_body,
    grid=(N // BN,),
    in_specs=[
        pl.BlockSpec((BN, D_IN), lambda i: (i, 0)),
        pl.BlockSpec((D_IN, H), lambda i: (0, 0)),
        pl.BlockSpec((1, H), lambda i: (0, 0)),
    ],
    out_specs=pl.BlockSpec((2, BN, HH), lambda i: (0, i, 0)),
    out_shape=jax.ShapeDtypeStruct((2, N, HH), jnp.float32),
)


def _round_body(s_ref, c_ref, h_ref, wl_ref, bl_ref, wr_ref, o_ref, *, relu):
  cnt = c_ref[0, :, 0:1] + c_ref[1, :, 0:1]
  inv = 1.0 / jnp.maximum(cnt, 1.0)
  f32 = jnp.float32
  o = (jnp.dot(s_ref[0] * inv, wl_ref[0], preferred_element_type=f32)
       + jnp.dot(s_ref[1] * inv, wl_ref[1], preferred_element_type=f32)
       + jnp.dot(h_ref[0], wr_ref[0], preferred_element_type=f32)
       + jnp.dot(h_ref[1], wr_ref[1], preferred_element_type=f32)
       + bl_ref[...])
  nrm = jnp.sqrt(jnp.sum(o * o, axis=1, keepdims=True))
  o = o / jnp.maximum(nrm, 1e-12)
  if relu:
    o = jnp.maximum(o, 0.0)
    o_ref[0] = o[:, :HH]
    o_ref[1] = o[:, HH:]
  else:
    o_ref[...] = o


_round_call = pl.pallas_call(
    functools.partial(_round_body, relu=True),
    grid=(N // BN,),
    in_specs=[
        pl.BlockSpec((2, BN, HH), lambda i: (0, i, 0)),
        pl.BlockSpec((2, BN, 16), lambda i: (0, i, 0)),
        pl.BlockSpec((2, BN, HH), lambda i: (0, i, 0)),
        pl.BlockSpec((2, HH, H), lambda i: (0, 0, 0)),
        pl.BlockSpec((1, H), lambda i: (0, 0)),
        pl.BlockSpec((2, HH, H), lambda i: (0, 0, 0)),
    ],
    out_specs=pl.BlockSpec((2, BN, HH), lambda i: (0, i, 0)),
    out_shape=jax.ShapeDtypeStruct((2, N, HH), jnp.float32),
)

_final_call = pl.pallas_call(
    functools.partial(_round_body, relu=False),
    grid=(N // BN,),
    in_specs=[
        pl.BlockSpec((2, BN, HH), lambda i: (0, i, 0)),
        pl.BlockSpec((2, BN, 16), lambda i: (0, i, 0)),
        pl.BlockSpec((2, BN, HH), lambda i: (0, i, 0)),
        pl.BlockSpec((2, HH, C), lambda i: (0, 0, 0)),
        pl.BlockSpec((1, C), lambda i: (0, 0)),
        pl.BlockSpec((2, HH, C), lambda i: (0, 0, 0)),
    ],
    out_specs=pl.BlockSpec((BN, C), lambda i: (i, 0)),
    out_shape=jax.ShapeDtypeStruct((N, C), jnp.float32),
)


def kernel(x, edge_index, edge_weight, W_emb, b_emb, Wl0, bl0, Wr0, Wl1, bl1,
           Wr1, Wl2, bl2, Wr2, Wlo, blo, Wro):
  pad = E_PAD - E
  src2 = jnp.concatenate(
      [edge_index[0].astype(jnp.int32), jnp.zeros((pad,), jnp.int32)]
  ).reshape(EDGE_ROWS, 128)
  src3 = jnp.stack([src2, src2 + N])
  dst2 = jnp.concatenate(
      [edge_index[1].astype(jnp.int32), jnp.full((pad,), N, jnp.int32)]
  ).reshape(EDGE_ROWS, 128)
  wpad = jnp.concatenate(
      [edge_weight.astype(jnp.float32), jnp.zeros((pad,), jnp.float32)])

  cnt2 = _count_kernel(dst2).reshape(2, N, 16)
  h2 = _emb_call(x, W_emb, b_emb.reshape(1, H))
  for Wl, bl, Wr in ((Wl0, bl0, Wr0), (Wl1, bl1, Wr1), (Wl2, bl2, Wr2)):
    s2 = _scatter_w(h2.reshape(2 * N, HH), src3, dst2, wpad).reshape(2, N, HH)
    h2 = _round_call(s2, cnt2, h2, Wl.reshape(2, HH, H), bl.reshape(1, H),
                     Wr.reshape(2, HH, H))
  s2 = _scatter_nw(h2.reshape(2 * N, HH), src3, dst2).reshape(2, N, HH)
  return _final_call(s2, cnt2, h2, Wlo.reshape(2, HH, C), blo.reshape(1, C),
                     Wro.reshape(2, HH, C))


# speedup vs baseline: 5.8984x; 1.4190x over previous
import functools

import jax
import jax.numpy as jnp
from jax import lax
from jax.experimental import pallas as pl
from jax.experimental.pallas import tpu as pltpu
from jax.experimental.pallas import tpu_sc as plsc

N = 50000
E = 800000
D_IN = 100
H = 64
HH = 32
C = 18

NUM_SC = 2
NUM_TILES = 16

E_PAD = 819200
EDGE_ROWS = E_PAD // 128
EPT = E_PAD // NUM_TILES
CHUNK = 256
NCH = EPT // CHUNK
SUB = 128
ACC_ROWS = 51200
ZPT = ACC_ROWS // NUM_TILES
DCH = 200
DCHUNKS = N // DCH

_mesh = plsc.VectorSubcoreMesh(core_axis_name="c", subcore_axis_name="s")


def _make_scatter(scale: bool):
  def body(*refs):
    if scale:
      (h_hbm, src_hbm, dst_hbm, w_hbm, out_hbm, acc, srcv, dstv, wv, rows,
       semg0, semg1, sems0, sems1, semi0, semi1) = refs
    else:
      (h_hbm, src_hbm, dst_hbm, out_hbm, acc, srcv, dstv, wv, rows,
       semg0, semg1, sems0, sems1, semi0, semi1) = refs
    semg = (semg0, semg1)
    sems = (sems0, sems1)
    semi = (semi0, semi1)
    cid = lax.axis_index("c")
    sid = lax.axis_index("s")
    zero16 = jnp.zeros((16,), jnp.float32)
    rbase = sid * (EPT // 128)
    wbase = sid * EPT
    coff = cid * N
    CR = CHUNK // 128

    for i in range(SUB):
      rows[0, i, pl.ds(0, 16)] = zero16
      rows[0, i, pl.ds(16, 16)] = zero16
    plsc.subcore_barrier()

    def zero_acc(i, carry):
      pltpu.sync_copy(rows.at[0, pl.ds(0, SUB)],
                      acc.at[pl.ds(sid * ZPT + i * SUB, SUB)])
      return carry

    lax.fori_loop(0, ZPT // SUB, zero_acc, 0)
    plsc.subcore_barrier()

    def issue_idx(c, p, d):
      pltpu.async_copy(src_hbm.at[cid, pl.ds(rbase + c * CR, CR)],
                       srcv.at[p], semi[p])
      pltpu.async_copy(dst_hbm.at[pl.ds(rbase + c * CR, CR)],
                       dstv.at[d], semi[p])
      if scale:
        pltpu.async_copy(w_hbm.at[pl.ds(wbase + c * CHUNK, CHUNK)],
                         wv.at[p], semi[p])

    def wait_idx(p, d):
      pltpu.make_async_copy(src_hbm.at[cid, pl.ds(rbase, CR)],
                            srcv.at[p], semi[p]).wait()
      pltpu.make_async_copy(dst_hbm.at[pl.ds(rbase, CR)],
                            dstv.at[d], semi[p]).wait()
      if scale:
        pltpu.make_async_copy(w_hbm.at[pl.ds(wbase, CHUNK)],
                              wv.at[p], semi[p]).wait()

    def issue_gather(p):
      for j in range(CR):
        pltpu.async_copy(h_hbm.at[srcv.at[p, j]],
                         rows.at[p, pl.ds(j * SUB, SUB)], semg[p])

    def wait_gather(p):
      for j in range(CR):
        pltpu.make_async_copy(h_hbm.at[srcv.at[p, j]],
                              rows.at[p, pl.ds(j * SUB, SUB)],
                              semg[p]).wait()

    def issue_scatter(p, d):
      for j in range(CR):
        pltpu.async_copy(rows.at[p, pl.ds(j * SUB, SUB)],
                         acc.at[dstv.at[d, j]], sems[p], add=True)

    def wait_scatter(p, d):
      for j in range(CR):
        pltpu.make_async_copy(rows.at[p, pl.ds(j * SUB, SUB)],
                              acc.at[dstv.at[d, j]], sems[p]).wait()

    def scale_rows(p):
      if scale:
        def grp(g, c2):
          w16 = wv[p, pl.ds(g * 16, 16)]
          for u in range(16):
            k = g * 16 + u
            w = w16[u]
            rows[p, k, pl.ds(0, 16)] = rows[p, k, pl.ds(0, 16)] * w
            rows[p, k, pl.ds(16, 16)] = rows[p, k, pl.ds(16, 16)] * w
          return c2

        lax.fori_loop(0, CHUNK // 16, grp, 0)
      plsc.subcore_barrier()

    def block(c, k, first=False, last=False, prefetch=True):
      p = k % 2
      if not first:
        wait_scatter(p, k)
      if not last:
        wait_idx(p, k)
        issue_gather(p)
      wait_gather(1 - p)
      scale_rows(1 - p)
      issue_scatter(1 - p, (k + 3) % 4)
      if prefetch:
        issue_idx(c + 1, 1 - p, (k + 1) % 4)

    pltpu.sync_copy(src_hbm.at[cid, pl.ds(rbase, CR)], srcv.at[0])
    pltpu.sync_copy(dst_hbm.at[pl.ds(rbase, CR)], dstv.at[0])
    if scale:
      pltpu.sync_copy(w_hbm.at[pl.ds(wbase, CHUNK)], wv.at[0])
    issue_gather(0)
    issue_idx(1, 1, 1)

    block(1, 1, first=True)

    def quad(i, carry):
      c = 4 * i + 2
      for k in range(4):
        block(c + k, (2 + k) % 4)
      return carry

    lax.fori_loop(0, (NCH - 2) // 4, quad, 0)
    block(NCH - 2, (NCH - 2) % 4)
    block(NCH - 1, (NCH - 1) % 4, prefetch=False)
    block(NCH, NCH % 4, last=True, prefetch=False)
    wait_scatter((NCH - 1) % 2, (NCH - 1) % 4)
    plsc.subcore_barrier()

    def drain(k, carry):
      idx = sid + k * NUM_TILES

      @pl.when(idx < DCHUNKS)
      def _():
        b = idx * DCH
        pltpu.sync_copy(acc.at[pl.ds(b, DCH)], rows.at[0, pl.ds(0, DCH)])
        pltpu.sync_copy(rows.at[0, pl.ds(0, DCH)],
                        out_hbm.at[pl.ds(coff + b, DCH)])

      return carry

    lax.fori_loop(0, pl.cdiv(DCHUNKS, NUM_TILES), drain, 0)

  return pl.kernel(
      body,
      out_type=jax.ShapeDtypeStruct((NUM_SC * N, HH), jnp.float32),
      mesh=_mesh,
      compiler_params=pltpu.CompilerParams(use_tc_tiling_on_sc=False),
      scratch_types=[
          pltpu.VMEM_SHARED((ACC_ROWS, HH), jnp.float32),
          pltpu.VMEM((2, CHUNK // 128, 128), jnp.int32),
          pltpu.VMEM((4, CHUNK // 128, 128), jnp.int32),
          pltpu.VMEM((2, CHUNK), jnp.float32),
          pltpu.VMEM((2, CHUNK, HH), jnp.float32),
          pltpu.SemaphoreType.DMA,
          pltpu.SemaphoreType.DMA,
          pltpu.SemaphoreType.DMA,
          pltpu.SemaphoreType.DMA,
          pltpu.SemaphoreType.DMA,
          pltpu.SemaphoreType.DMA,
      ],
  )


def _count_body(dst_hbm, out_hbm, acc, dstv, ones, buf):
  cid = lax.axis_index("c")
  sid = lax.axis_index("s")
  zero16 = jnp.zeros((16,), jnp.float32)
  one16 = jnp.ones((16,), jnp.float32)
  for i in range(SUB):
    ones[i, pl.ds(0, 16)] = one16
  for i in range(SUB):
    buf[i, pl.ds(0, 16)] = zero16
  plsc.subcore_barrier()

  def zero_acc(i, carry):
    pltpu.sync_copy(buf.at[pl.ds(0, SUB)],
                    acc.at[pl.ds(sid * ZPT + i * SUB, SUB)])
    return carry

  lax.fori_loop(0, ZPT // SUB, zero_acc, 0)
  plsc.subcore_barrier()

  rows_per_tile = EDGE_ROWS // (NUM_SC * NUM_TILES)

  def chunk(i, carry):
    r0 = cid * (EDGE_ROWS // NUM_SC) + sid * rows_per_tile + i * 8
    pltpu.sync_copy(dst_hbm.at[pl.ds(r0, 8)], dstv)
    for j in range(8):
      pltpu.sync_copy(ones, acc.at[dstv.at[j]], add=True)
    return carry

  lax.fori_loop(0, rows_per_tile // 8, chunk, 0)
  plsc.subcore_barrier()

  def drain(k, carry):
    idx = sid + k * NUM_TILES

    @pl.when(idx < DCHUNKS)
    def _():
      b = idx * DCH
      pltpu.sync_copy(acc.at[pl.ds(b, DCH)], buf.at[pl.ds(0, DCH)])
      pltpu.sync_copy(buf.at[pl.ds(0, DCH)],
                      out_hbm.at[pl.ds(cid * N + b, DCH)])

    return carry

  lax.fori_loop(0, pl.cdiv(DCHUNKS, NUM_TILES), drain, 0)


_count_kernel = pl.kernel(
    _count_body,
    out_type=jax.ShapeDtypeStruct((NUM_SC * N, 16), jnp.float32),
    mesh=_mesh,
    compiler_params=pltpu.CompilerParams(use_tc_tiling_on_sc=False),
    scratch_types=[
        pltpu.VMEM_SHARED((ACC_ROWS, 16), jnp.float32),
        pltpu.VMEM((8, 128), jnp.int32),
        pltpu.VMEM((SUB, 16), jnp.float32),
        pltpu.VMEM((DCH, 16), jnp.float32),
    ],
)

_scatter_w = _make_scatter(True)
_scatter_nw = _make_scatter(False)

BN = 2000


def _emb_body(x_ref, w_ref, b_ref, o_ref):
  h = jnp.dot(x_ref[...], w_ref[...], preferred_element_type=jnp.float32)
  h = jnp.maximum(h + b_ref[...], 0.0)
  o_ref[0] = h[:, :HH]
  o_ref[1] = h[:, HH:]


_emb_call = pl.pallas_call(
    _emb_body,
    grid=(N // BN,),
    in_specs=[
        pl.BlockSpec((BN, D_IN), lambda i: (i, 0)),
        pl.BlockSpec((D_IN, H), lambda i: (0, 0)),
        pl.BlockSpec((1, H), lambda i: (0, 0)),
    ],
    out_specs=pl.BlockSpec((2, BN, HH), lambda i: (0, i, 0)),
    out_shape=jax.ShapeDtypeStruct((2, N, HH), jnp.float32),
)


def _round_body(s_ref, c_ref, h_ref, wl_ref, bl_ref, wr_ref, o_ref, *, relu):
  cnt = c_ref[0, :, 0:1] + c_ref[1, :, 0:1]
  inv = 1.0 / jnp.maximum(cnt, 1.0)
  f32 = jnp.float32
  o = (jnp.dot(s_ref[0] * inv, wl_ref[0], preferred_element_type=f32)
       + jnp.dot(s_ref[1] * inv, wl_ref[1], preferred_element_type=f32)
       + jnp.dot(h_ref[0], wr_ref[0], preferred_element_type=f32)
       + jnp.dot(h_ref[1], wr_ref[1], preferred_element_type=f32)
       + bl_ref[...])
  nrm = jnp.sqrt(jnp.sum(o * o, axis=1, keepdims=True))
  o = o / jnp.maximum(nrm, 1e-12)
  if relu:
    o = jnp.maximum(o, 0.0)
    o_ref[0] = o[:, :HH]
    o_ref[1] = o[:, HH:]
  else:
    o_ref[...] = o


_round_call = pl.pallas_call(
    functools.partial(_round_body, relu=True),
    grid=(N // BN,),
    in_specs=[
        pl.BlockSpec((2, BN, HH), lambda i: (0, i, 0)),
        pl.BlockSpec((2, BN, 16), lambda i: (0, i, 0)),
        pl.BlockSpec((2, BN, HH), lambda i: (0, i, 0)),
        pl.BlockSpec((2, HH, H), lambda i: (0, 0, 0)),
        pl.BlockSpec((1, H), lambda i: (0, 0)),
        pl.BlockSpec((2, HH, H), lambda i: (0, 0, 0)),
    ],
    out_specs=pl.BlockSpec((2, BN, HH), lambda i: (0, i, 0)),
    out_shape=jax.ShapeDtypeStruct((2, N, HH), jnp.float32),
)

_final_call = pl.pallas_call(
    functools.partial(_round_body, relu=False),
    grid=(N // BN,),
    in_specs=[
        pl.BlockSpec((2, BN, HH), lambda i: (0, i, 0)),
        pl.BlockSpec((2, BN, 16), lambda i: (0, i, 0)),
        pl.BlockSpec((2, BN, HH), lambda i: (0, i, 0)),
        pl.BlockSpec((2, HH, C), lambda i: (0, 0, 0)),
        pl.BlockSpec((1, C), lambda i: (0, 0)),
        pl.BlockSpec((2, HH, C), lambda i: (0, 0, 0)),
    ],
    out_specs=pl.BlockSpec((BN, C), lambda i: (i, 0)),
    out_shape=jax.ShapeDtypeStruct((N, C), jnp.float32),
)


def kernel(x, edge_index, edge_weight, W_emb, b_emb, Wl0, bl0, Wr0, Wl1, bl1,
           Wr1, Wl2, bl2, Wr2, Wlo, blo, Wro):
  pad = E_PAD - E
  src2 = jnp.concatenate(
      [edge_index[0].astype(jnp.int32), jnp.zeros((pad,), jnp.int32)]
  ).reshape(EDGE_ROWS, 128)
  src3 = jnp.stack([src2, src2 + N])
  dst2 = jnp.concatenate(
      [edge_index[1].astype(jnp.int32), jnp.full((pad,), N, jnp.int32)]
  ).reshape(EDGE_ROWS, 128)
  wpad = jnp.concatenate(
      [edge_weight.astype(jnp.float32), jnp.zeros((pad,), jnp.float32)])

  cnt2 = _count_kernel(dst2).reshape(2, N, 16)
  h2 = _emb_call(x, W_emb, b_emb.reshape(1, H))
  for Wl, bl, Wr in ((Wl0, bl0, Wr0), (Wl1, bl1, Wr1), (Wl2, bl2, Wr2)):
    s2 = _scatter_w(h2.reshape(2 * N, HH), src3, dst2, wpad).reshape(2, N, HH)
    h2 = _round_call(s2, cnt2, h2, Wl.reshape(2, HH, H), bl.reshape(1, H),
                     Wr.reshape(2, HH, H))
  s2 = _scatter_nw(h2.reshape(2 * N, HH), src3, dst2).reshape(2, N, HH)
  return _final_call(s2, cnt2, h2, Wlo.reshape(2, HH, C), blo.reshape(1, C),
                     Wro.reshape(2, HH, C))
